# baseline (speedup 1.0000x reference)
import functools

import jax
import jax.numpy as jnp
from jax import lax
from jax.experimental import pallas as pl
from jax.experimental.pallas import tpu as pltpu
from jax.experimental.pallas import tpu_sc as plsc

N = 10000
E = 320000
D = 128
H = 128
G = 64
C = 10

NC = 2
NS = 16
NW = NC * NS
EPT = E // NW
BB = 128
NB = 80
EPAD = NB * BB
NPAD = 10240
RPT = NPAD // NS
TRASH = N

_f32 = jnp.float32


def _deg_body(dstp, degp, idx_v, ones_v, zb_v, acc_sp):
    c = lax.axis_index("c")
    s = lax.axis_index("s")
    wid = c * NS + s
    pltpu.sync_copy(dstp.at[wid], idx_v)
    for k in range(8):
        ones_v[pl.ds(k * 16, 16)] = jnp.ones((16,), _f32)

    def zf(i, _):
        zb_v[pl.ds(i * 16, 16)] = jnp.zeros((16,), _f32)
        return 0
    lax.fori_loop(0, RPT // 16, zf, 0)
    pltpu.sync_copy(zb_v, acc_sp.at[pl.ds(s * RPT, RPT)])
    plsc.subcore_barrier()

    def eb(j, _):
        pltpu.sync_copy(ones_v, acc_sp.at[idx_v.at[j]], add=True)
        return 0
    lax.fori_loop(0, NB, eb, 0)
    plsc.subcore_barrier()
    pltpu.sync_copy(acc_sp.at[pl.ds(s * RPT, RPT)], degp.at[c, pl.ds(s * RPT, RPT)])


@functools.lru_cache(maxsize=None)
def _sc_calls():
    mesh = plsc.VectorSubcoreMesh(core_axis_name="c", subcore_axis_name="s")
    deg = pl.kernel(
        _deg_body,
        out_type=jax.ShapeDtypeStruct((NC, NPAD), _f32),
        mesh=mesh,
        scratch_types=[
            pltpu.VMEM((NB, BB), jnp.int32),
            pltpu.VMEM((BB,), _f32),
            pltpu.VMEM((RPT,), _f32),
            pltpu.VMEM_SHARED((NPAD,), _f32),
        ],
    )
    prop = pl.kernel(
        _prop_body,
        out_type=jax.ShapeDtypeStruct((NC, NPAD, H), _f32),
        mesh=mesh,
        scratch_types=[
            pltpu.VMEM((2, WW, BB), jnp.int32),
            pltpu.VMEM((2, WW, BB), jnp.int32),
            pltpu.VMEM((BB, H), _f32),
            pltpu.VMEM((BB, H), _f32),
            pltpu.VMEM((32, H), _f32),
            pltpu.SemaphoreType.DMA,
            pltpu.SemaphoreType.DMA,
            pltpu.SemaphoreType.DMA,
            pltpu.SemaphoreType.DMA,
            pltpu.VMEM_SHARED((NPAD, H), _f32),
        ],
    )
    return deg, prop


WW = 16
NWIN = NB // WW


def _prop_body(y, srcp, dstp, outp, iws, iwd, rows0, rows1, zb,
               is0, is1, gs0, gs1, acc_sp):
    c = lax.axis_index("c")
    s = lax.axis_index("s")
    wid = c * NS + s
    isems = (is0, is1)

    def iw_start(w, bi):
        pltpu.async_copy(srcp.at[wid, pl.ds(w * WW, WW)], iws.at[bi], isems[bi])
        pltpu.async_copy(dstp.at[wid, pl.ds(w * WW, WW)], iwd.at[bi], isems[bi])

    def iw_wait(w, bi):
        pltpu.make_async_copy(srcp.at[wid, pl.ds(w * WW, WW)], iws.at[bi],
                              isems[bi]).wait()
        pltpu.make_async_copy(dstp.at[wid, pl.ds(w * WW, WW)], iwd.at[bi],
                              isems[bi]).wait()

    iw_start(0, 0)

    def zf(i, _):
        for k in range(8):
            zb[i, pl.ds(k * 16, 16)] = jnp.zeros((16,), _f32)
        return 0
    lax.fori_loop(0, 32, zf, 0)

    def zcp(i, _):
        pltpu.sync_copy(zb, acc_sp.at[pl.ds(s * RPT + i * 32, 32)])
        return 0
    lax.fori_loop(0, RPT // 32, zcp, 0)
    plsc.subcore_barrier()

    def g_start(bi, lj, buf, sem):
        pltpu.async_copy(y.at[iws.at[bi, lj]], buf, sem)

    def g_wait(bi, lj, buf, sem):
        pltpu.make_async_copy(y.at[iws.at[bi, lj]], buf, sem).wait()

    def s_sync(bi, lj, buf):
        pltpu.sync_copy(buf, acc_sp.at[iwd.at[bi, lj]], add=True)

    for w in range(NWIN):
        bi = w % 2
        if w + 1 < NWIN:
            iw_start(w + 1, 1 - bi)
        iw_wait(w, bi)
        g_start(bi, 0, rows0, gs0)

        def pair(kb, _):
            lj0 = 2 * kb
            lj1 = lj0 + 1
            ljn = jnp.minimum(lj1 + 1, WW - 1)
            g_wait(bi, lj0, rows0, gs0)
            g_start(bi, lj1, rows1, gs1)
            s_sync(bi, lj0, rows0)
            g_wait(bi, lj1, rows1, gs1)
            g_start(bi, ljn, rows0, gs0)
            s_sync(bi, lj1, rows1)
            return 0
        lax.fori_loop(0, WW // 2, pair, 0)
        g_wait(bi, WW - 1, rows0, gs0)
    plsc.subcore_barrier()
    pltpu.sync_copy(acc_sp.at[pl.ds(s * RPT, RPT)],
                    outp.at[c, pl.ds(s * RPT, RPT)])


BLK = 1000
NBLK = N // BLK


def _mm_body(x_ref, w_ref, o_ref):
    o_ref[...] = jnp.dot(x_ref[...], w_ref[...],
                         preferred_element_type=_f32)


_mm1 = pl.pallas_call(
    _mm_body,
    grid=(NBLK,),
    in_specs=[pl.BlockSpec((BLK, D), lambda i: (i, 0)),
              pl.BlockSpec((D, H), lambda i: (0, 0))],
    out_specs=pl.BlockSpec((BLK, H), lambda i: (i, 0)),
    out_shape=jax.ShapeDtypeStruct((N, H), _f32),
)


def _t0_body(degp_ref, xw_ref, dis_ref, dis2_ref, y_ref):
    deg = degp_ref[0] + degp_ref[1] + 1.0
    dis = lax.rsqrt(deg)
    dis2 = 1.0 / deg
    dis_ref[...] = dis
    dis2_ref[...] = dis2
    y_ref[...] = xw_ref[...] * dis


_t0 = pl.pallas_call(
    _t0_body,
    grid=(NBLK,),
    in_specs=[pl.BlockSpec((NC, BLK, 1), lambda i: (0, i, 0)),
              pl.BlockSpec((BLK, H), lambda i: (i, 0))],
    out_specs=[pl.BlockSpec((BLK, 1), lambda i: (i, 0)),
               pl.BlockSpec((BLK, 1), lambda i: (i, 0)),
               pl.BlockSpec((BLK, H), lambda i: (i, 0))],
    out_shape=[jax.ShapeDtypeStruct((N, 1), _f32),
               jax.ShapeDtypeStruct((N, 1), _f32),
               jax.ShapeDtypeStruct((N, H), _f32)],
)


def _blayer_body(with_next, with_cnt, *refs):
    if with_next:
        (acc_ref, xw_ref, dis_ref, dis2_ref, b_ref, bat_ref, w_ref,
         s_ref, *rest) = refs
        if with_cnt:
            cnt_ref, xwn_ref, yn_ref = rest
        else:
            xwn_ref, yn_ref = rest
    else:
        acc_ref, xw_ref, dis_ref, dis2_ref, b_ref, bat_ref, s_ref = refs
    i = pl.program_id(0)
    dis = dis_ref[...]
    a = acc_ref[0] + acc_ref[1]
    h = jnp.maximum(a * dis + xw_ref[...] * dis2_ref[...] + b_ref[...], 0.0)
    bat = bat_ref[0]
    gi = lax.broadcasted_iota(jnp.int32, (G, BLK), 0)
    oh = (gi == bat).astype(_f32)
    sc = jnp.dot(oh, h, preferred_element_type=_f32)

    @pl.when(i == 0)
    def _():
        s_ref[...] = jnp.zeros_like(s_ref)
        if with_next and with_cnt:
            cnt_ref[...] = jnp.zeros_like(cnt_ref)

    s_ref[...] += sc
    if with_next:
        if with_cnt:
            cnt_ref[...] += jnp.sum(oh, axis=1, keepdims=True)
        xwn = jnp.dot(h, w_ref[...], preferred_element_type=_f32)
        xwn_ref[...] = xwn
        yn_ref[...] = xwn * dis


def _make_blayer(with_next, with_cnt):
    in_specs = [
        pl.BlockSpec((NC, BLK, H), lambda i: (0, i, 0)),
        pl.BlockSpec((BLK, H), lambda i: (i, 0)),
        pl.BlockSpec((BLK, 1), lambda i: (i, 0)),
        pl.BlockSpec((BLK, 1), lambda i: (i, 0)),
        pl.BlockSpec((1, H), lambda i: (0, 0)),
        pl.BlockSpec((1, 1, BLK), lambda i: (i, 0, 0)),
    ]
    out_specs = [pl.BlockSpec((G, H), lambda i: (0, 0))]
    out_shape = [jax.ShapeDtypeStruct((G, H), _f32)]
    if with_next:
        in_specs.append(pl.BlockSpec((H, H), lambda i: (0, 0)))
        if with_cnt:
            out_specs.append(pl.BlockSpec((G, 1), lambda i: (0, 0)))
            out_shape.append(jax.ShapeDtypeStruct((G, 1), _f32))
        out_specs += [pl.BlockSpec((BLK, H), lambda i: (i, 0)),
                      pl.BlockSpec((BLK, H), lambda i: (i, 0))]
        out_shape += [jax.ShapeDtypeStruct((N, H), _f32),
                      jax.ShapeDtypeStruct((N, H), _f32)]
    return pl.pallas_call(
        functools.partial(_blayer_body, with_next, with_cnt),
        grid=(NBLK,),
        in_specs=in_specs,
        out_specs=out_specs,
        out_shape=out_shape,
    )


_b_first = _make_blayer(True, True)
_b_mid = _make_blayer(True, False)
_b_last = _make_blayer(False, False)


def _head_body(s1_ref, s2_ref, s3_ref, cnt_ref, pc_ref,
               lw1_ref, lb1_ref, lw2_ref, lb2_ref, lw3_ref, lb3_ref, o_ref):
    inv = 1.0 / jnp.maximum(cnt_ref[...], 1.0)
    g = (jnp.maximum(s1_ref[...] * inv, 0.0)
         + jnp.maximum(s2_ref[...] * inv, 0.0)
         + jnp.maximum(s3_ref[...] * inv, 0.0))
    g1 = jnp.maximum(
        jnp.dot(g, lw1_ref[...], preferred_element_type=_f32) + lb1_ref[...],
        0.0)
    l2 = lw2_ref[...]
    g2 = jnp.maximum(
        jnp.dot(g1, l2[:H // 2], preferred_element_type=_f32)
        + pc_ref[...] * l2[H // 2:H // 2 + 1]
        + lb2_ref[...],
        0.0)
    z = jnp.dot(g2, lw3_ref[...], preferred_element_type=_f32) + lb3_ref[...]
    m = jnp.max(z, axis=-1, keepdims=True)
    e = jnp.exp(z - m)
    o_ref[...] = z - m - jnp.log(jnp.sum(e, axis=-1, keepdims=True))


_head = pl.pallas_call(
    _head_body,
    grid=(1,),
    in_specs=[pl.BlockSpec((G, H), lambda i: (0, 0)),
              pl.BlockSpec((G, H), lambda i: (0, 0)),
              pl.BlockSpec((G, H), lambda i: (0, 0)),
              pl.BlockSpec((G, 1), lambda i: (0, 0)),
              pl.BlockSpec((G, 1), lambda i: (0, 0)),
              pl.BlockSpec((H, H // 2), lambda i: (0, 0)),
              pl.BlockSpec((1, H // 2), lambda i: (0, 0)),
              pl.BlockSpec((H // 2 + 1, H // 4), lambda i: (0, 0)),
              pl.BlockSpec((1, H // 4), lambda i: (0, 0)),
              pl.BlockSpec((H // 4, C), lambda i: (0, 0)),
              pl.BlockSpec((1, C), lambda i: (0, 0))],
    out_specs=pl.BlockSpec((G, C), lambda i: (0, 0)),
    out_shape=jax.ShapeDtypeStruct((G, C), _f32),
)


def kernel(x, edge_index, batch, paper_count, W1, b1, W2, b2, W3, b3,
           lw1, lb1, lw2, lb2, lw3, lb3):
    pad = EPAD - EPT
    src = edge_index[0].reshape(NW, EPT)
    dst = edge_index[1].reshape(NW, EPT)
    srcp = jnp.concatenate(
        [src, jnp.zeros((NW, pad), jnp.int32)], axis=1).reshape(NW, NB, BB)
    dstp = jnp.concatenate(
        [dst, jnp.full((NW, pad), TRASH, jnp.int32)], axis=1).reshape(NW, NB, BB)
    bat3 = batch.reshape(NBLK, 1, BLK)
    _deg_call, _prop_call = _sc_calls()

    degp = _deg_call(dstp).reshape(NC, NPAD, 1)
    xw1 = _mm1(x, W1)
    dis, dis2, y1 = _t0(degp, xw1)

    acc1 = _prop_call(y1, srcp, dstp)
    s1, cnt, xw2, y2 = _b_first(acc1, xw1, dis, dis2, b1.reshape(1, H),
                                bat3, W2)
    acc2 = _prop_call(y2, srcp, dstp)
    s2, xw3, y3 = _b_mid(acc2, xw2, dis, dis2, b2.reshape(1, H), bat3, W3)
    acc3 = _prop_call(y3, srcp, dstp)
    s3 = _b_last(acc3, xw3, dis, dis2, b3.reshape(1, H), bat3)
    if isinstance(s3, (list, tuple)):
        s3 = s3[0]

    return _head(s1, s2, s3, cnt, paper_count.reshape(G, 1),
                 lw1, lb1.reshape(1, H // 2), lw2, lb2.reshape(1, H // 4),
                 lw3, lb3.reshape(1, C))


# speedup vs baseline: 1.2351x; 1.2351x over previous
import functools

import jax
import jax.numpy as jnp
from jax import lax
from jax.experimental import pallas as pl
from jax.experimental.pallas import tpu as pltpu
from jax.experimental.pallas import tpu_sc as plsc

N = 10000
E = 320000
D = 128
H = 128
G = 64
C = 10

NC = 2
NS = 16
NW = NC * NS
EPT = E // NW
BB = 128
NB = 80
EPAD = NB * BB
NPAD = 10240
RPT = NPAD // NS
TRASH = N

_f32 = jnp.float32


def _deg_body(dstp, degp, idx_v, ones_v, zb_v, acc_sp):
    c = lax.axis_index("c")
    s = lax.axis_index("s")
    wid = c * NS + s
    pltpu.sync_copy(dstp.at[wid], idx_v)
    for k in range(8):
        ones_v[pl.ds(k * 16, 16)] = jnp.ones((16,), _f32)

    def zf(i, _):
        zb_v[pl.ds(i * 16, 16)] = jnp.zeros((16,), _f32)
        return 0
    lax.fori_loop(0, RPT // 16, zf, 0)
    pltpu.sync_copy(zb_v, acc_sp.at[pl.ds(s * RPT, RPT)])
    plsc.subcore_barrier()

    def eb(j, _):
        pltpu.sync_copy(ones_v, acc_sp.at[idx_v.at[j]], add=True)
        return 0
    lax.fori_loop(0, NB, eb, 0)
    plsc.subcore_barrier()
    pltpu.sync_copy(acc_sp.at[pl.ds(s * RPT, RPT)], degp.at[c, pl.ds(s * RPT, RPT)])


@functools.lru_cache(maxsize=None)
def _sc_calls():
    mesh = plsc.VectorSubcoreMesh(core_axis_name="c", subcore_axis_name="s")
    deg = pl.kernel(
        _deg_body,
        out_type=jax.ShapeDtypeStruct((NC, NPAD), _f32),
        mesh=mesh,
        scratch_types=[
            pltpu.VMEM((NB, BB), jnp.int32),
            pltpu.VMEM((BB,), _f32),
            pltpu.VMEM((RPT,), _f32),
            pltpu.VMEM_SHARED((NPAD,), _f32),
        ],
    )
    prop = pl.kernel(
        _prop_body,
        out_type=jax.ShapeDtypeStruct((NC, NPAD, H), _f32),
        mesh=mesh,
        scratch_types=[
            pltpu.VMEM((NB, BB), jnp.int32),
            pltpu.VMEM((NB, BB), jnp.int32),
            pltpu.VMEM((BB, H), _f32),
            pltpu.VMEM((64, H), _f32),
            pltpu.VMEM_SHARED((NPAD, H), _f32),
        ],
    )
    return deg, prop


def _prop_body(y, srcp, dstp, outp, sidx, didx, rows, zb, acc_sp):
    c = lax.axis_index("c")
    s = lax.axis_index("s")
    wid = c * NS + s
    pltpu.sync_copy(srcp.at[wid], sidx)
    pltpu.sync_copy(dstp.at[wid], didx)

    def zf(i, _):
        for k in range(8):
            zb[i, pl.ds(k * 16, 16)] = jnp.zeros((16,), _f32)
        return 0
    lax.fori_loop(0, 64, zf, 0)

    def zcp(i, _):
        pltpu.sync_copy(zb, acc_sp.at[pl.ds(s * RPT + i * 64, 64)])
        return 0
    lax.fori_loop(0, RPT // 64, zcp, 0)
    plsc.subcore_barrier()

    def eb(j, _):
        pltpu.sync_copy(y.at[sidx.at[j]], rows)
        pltpu.sync_copy(rows, acc_sp.at[didx.at[j]], add=True)
        return 0
    lax.fori_loop(0, NB, eb, 0)
    plsc.subcore_barrier()
    pltpu.sync_copy(acc_sp.at[pl.ds(s * RPT, RPT)],
                    outp.at[c, pl.ds(s * RPT, RPT)])


BLK = 1000
NBLK = N // BLK


def _mm_body(x_ref, w_ref, o_ref):
    o_ref[...] = jnp.dot(x_ref[...], w_ref[...],
                         preferred_element_type=_f32)


_mm1 = pl.pallas_call(
    _mm_body,
    grid=(NBLK,),
    in_specs=[pl.BlockSpec((BLK, D), lambda i: (i, 0)),
              pl.BlockSpec((D, H), lambda i: (0, 0))],
    out_specs=pl.BlockSpec((BLK, H), lambda i: (i, 0)),
    out_shape=jax.ShapeDtypeStruct((N, H), _f32),
)


def _t0_body(degp_ref, xw_ref, dis_ref, dis2_ref, y_ref):
    deg = degp_ref[0] + degp_ref[1] + 1.0
    dis = lax.rsqrt(deg)
    dis2 = 1.0 / deg
    dis_ref[...] = dis
    dis2_ref[...] = dis2
    y_ref[...] = xw_ref[...] * dis


_t0 = pl.pallas_call(
    _t0_body,
    grid=(NBLK,),
    in_specs=[pl.BlockSpec((NC, BLK, 1), lambda i: (0, i, 0)),
              pl.BlockSpec((BLK, H), lambda i: (i, 0))],
    out_specs=[pl.BlockSpec((BLK, 1), lambda i: (i, 0)),
               pl.BlockSpec((BLK, 1), lambda i: (i, 0)),
               pl.BlockSpec((BLK, H), lambda i: (i, 0))],
    out_shape=[jax.ShapeDtypeStruct((N, 1), _f32),
               jax.ShapeDtypeStruct((N, 1), _f32),
               jax.ShapeDtypeStruct((N, H), _f32)],
)


def _blayer_body(with_next, with_cnt, *refs):
    if with_next:
        (acc_ref, xw_ref, dis_ref, dis2_ref, b_ref, bat_ref, w_ref,
         s_ref, *rest) = refs
        if with_cnt:
            cnt_ref, xwn_ref, yn_ref = rest
        else:
            xwn_ref, yn_ref = rest
    else:
        acc_ref, xw_ref, dis_ref, dis2_ref, b_ref, bat_ref, s_ref = refs
    i = pl.program_id(0)
    dis = dis_ref[...]
    a = acc_ref[0] + acc_ref[1]
    h = jnp.maximum(a * dis + xw_ref[...] * dis2_ref[...] + b_ref[...], 0.0)
    bat = bat_ref[0]
    gi = lax.broadcasted_iota(jnp.int32, (G, BLK), 0)
    oh = (gi == bat).astype(_f32)
    sc = jnp.dot(oh, h, preferred_element_type=_f32)

    @pl.when(i == 0)
    def _():
        s_ref[...] = jnp.zeros_like(s_ref)
        if with_next and with_cnt:
            cnt_ref[...] = jnp.zeros_like(cnt_ref)

    s_ref[...] += sc
    if with_next:
        if with_cnt:
            cnt_ref[...] += jnp.sum(oh, axis=1, keepdims=True)
        xwn = jnp.dot(h, w_ref[...], preferred_element_type=_f32)
        xwn_ref[...] = xwn
        yn_ref[...] = xwn * dis


def _make_blayer(with_next, with_cnt):
    in_specs = [
        pl.BlockSpec((NC, BLK, H), lambda i: (0, i, 0)),
        pl.BlockSpec((BLK, H), lambda i: (i, 0)),
        pl.BlockSpec((BLK, 1), lambda i: (i, 0)),
        pl.BlockSpec((BLK, 1), lambda i: (i, 0)),
        pl.BlockSpec((1, H), lambda i: (0, 0)),
        pl.BlockSpec((1, 1, BLK), lambda i: (i, 0, 0)),
    ]
    out_specs = [pl.BlockSpec((G, H), lambda i: (0, 0))]
    out_shape = [jax.ShapeDtypeStruct((G, H), _f32)]
    if with_next:
        in_specs.append(pl.BlockSpec((H, H), lambda i: (0, 0)))
        if with_cnt:
            out_specs.append(pl.BlockSpec((G, 1), lambda i: (0, 0)))
            out_shape.append(jax.ShapeDtypeStruct((G, 1), _f32))
        out_specs += [pl.BlockSpec((BLK, H), lambda i: (i, 0)),
                      pl.BlockSpec((BLK, H), lambda i: (i, 0))]
        out_shape += [jax.ShapeDtypeStruct((N, H), _f32),
                      jax.ShapeDtypeStruct((N, H), _f32)]
    return pl.pallas_call(
        functools.partial(_blayer_body, with_next, with_cnt),
        grid=(NBLK,),
        in_specs=in_specs,
        out_specs=out_specs,
        out_shape=out_shape,
    )


_b_first = _make_blayer(True, True)
_b_mid = _make_blayer(True, False)
_b_last = _make_blayer(False, False)


def _head_body(s1_ref, s2_ref, s3_ref, cnt_ref, pc_ref,
               lw1_ref, lb1_ref, lw2_ref, lb2_ref, lw3_ref, lb3_ref, o_ref):
    inv = 1.0 / jnp.maximum(cnt_ref[...], 1.0)
    g = (jnp.maximum(s1_ref[...] * inv, 0.0)
         + jnp.maximum(s2_ref[...] * inv, 0.0)
         + jnp.maximum(s3_ref[...] * inv, 0.0))
    g1 = jnp.maximum(
        jnp.dot(g, lw1_ref[...], preferred_element_type=_f32) + lb1_ref[...],
        0.0)
    l2 = lw2_ref[...]
    g2 = jnp.maximum(
        jnp.dot(g1, l2[:H // 2], preferred_element_type=_f32)
        + pc_ref[...] * l2[H // 2:H // 2 + 1]
        + lb2_ref[...],
        0.0)
    z = jnp.dot(g2, lw3_ref[...], preferred_element_type=_f32) + lb3_ref[...]
    m = jnp.max(z, axis=-1, keepdims=True)
    e = jnp.exp(z - m)
    o_ref[...] = z - m - jnp.log(jnp.sum(e, axis=-1, keepdims=True))


_head = pl.pallas_call(
    _head_body,
    grid=(1,),
    in_specs=[pl.BlockSpec((G, H), lambda i: (0, 0)),
              pl.BlockSpec((G, H), lambda i: (0, 0)),
              pl.BlockSpec((G, H), lambda i: (0, 0)),
              pl.BlockSpec((G, 1), lambda i: (0, 0)),
              pl.BlockSpec((G, 1), lambda i: (0, 0)),
              pl.BlockSpec((H, H // 2), lambda i: (0, 0)),
              pl.BlockSpec((1, H // 2), lambda i: (0, 0)),
              pl.BlockSpec((H // 2 + 1, H // 4), lambda i: (0, 0)),
              pl.BlockSpec((1, H // 4), lambda i: (0, 0)),
              pl.BlockSpec((H // 4, C), lambda i: (0, 0)),
              pl.BlockSpec((1, C), lambda i: (0, 0))],
    out_specs=pl.BlockSpec((G, C), lambda i: (0, 0)),
    out_shape=jax.ShapeDtypeStruct((G, C), _f32),
)


def kernel(x, edge_index, batch, paper_count, W1, b1, W2, b2, W3, b3,
           lw1, lb1, lw2, lb2, lw3, lb3):
    pad = EPAD - EPT
    src = edge_index[0].reshape(NW, EPT)
    dst = edge_index[1].reshape(NW, EPT)
    srcp = jnp.concatenate(
        [src, jnp.zeros((NW, pad), jnp.int32)], axis=1).reshape(NW, NB, BB)
    dstp = jnp.concatenate(
        [dst, jnp.full((NW, pad), TRASH, jnp.int32)], axis=1).reshape(NW, NB, BB)
    bat3 = batch.reshape(NBLK, 1, BLK)
    _deg_call, _prop_call = _sc_calls()

    degp = _deg_call(dstp).reshape(NC, NPAD, 1)
    xw1 = _mm1(x, W1)
    dis, dis2, y1 = _t0(degp, xw1)

    acc1 = _prop_call(y1, srcp, dstp)
    s1, cnt, xw2, y2 = _b_first(acc1, xw1, dis, dis2, b1.reshape(1, H),
                                bat3, W2)
    acc2 = _prop_call(y2, srcp, dstp)
    s2, xw3, y3 = _b_mid(acc2, xw2, dis, dis2, b2.reshape(1, H), bat3, W3)
    acc3 = _prop_call(y3, srcp, dstp)
    s3 = _b_last(acc3, xw3, dis, dis2, b3.reshape(1, H), bat3)
    if isinstance(s3, (list, tuple)):
        s3 = s3[0]

    return _head(s1, s2, s3, cnt, paper_count.reshape(G, 1),
                 lw1, lb1.reshape(1, H // 2), lw2, lb2.reshape(1, H // 4),
                 lw3, lb3.reshape(1, C))


# speedup vs baseline: 2.8624x; 2.3174x over previous
import functools

import jax
import jax.numpy as jnp
from jax import lax
from jax.experimental import pallas as pl
from jax.experimental.pallas import tpu as pltpu
from jax.experimental.pallas import tpu_sc as plsc

N = 10000
E = 320000
D = 128
H = 128
G = 64
C = 10

NC = 2
NS = 16
NW = NC * NS
EPT = E // NW
BB = 128
NB = 80
EPAD = NB * BB
NPAD = 10240
RPT = NPAD // NS
TRASH = N

_f32 = jnp.float32


def _deg_body(dstp, degp, idx_v, ones_v, zb_v, acc_sp):
    c = lax.axis_index("c")
    s = lax.axis_index("s")
    wid = c * NS + s
    pltpu.sync_copy(dstp.at[wid], idx_v)
    for k in range(8):
        ones_v[pl.ds(k * 16, 16)] = jnp.ones((16,), _f32)

    def zf(i, _):
        zb_v[pl.ds(i * 16, 16)] = jnp.zeros((16,), _f32)
        return 0
    lax.fori_loop(0, RPT // 16, zf, 0)
    pltpu.sync_copy(zb_v, acc_sp.at[pl.ds(s * RPT, RPT)])
    plsc.subcore_barrier()

    def eb(j, _):
        pltpu.sync_copy(ones_v, acc_sp.at[idx_v.at[j]], add=True)
        return 0
    lax.fori_loop(0, NB, eb, 0)
    plsc.subcore_barrier()
    pltpu.sync_copy(acc_sp.at[pl.ds(s * RPT, RPT)], degp.at[c, pl.ds(s * RPT, RPT)])


@functools.lru_cache(maxsize=None)
def _sc_calls():
    mesh = plsc.VectorSubcoreMesh(core_axis_name="c", subcore_axis_name="s")
    deg = pl.kernel(
        _deg_body,
        out_type=jax.ShapeDtypeStruct((NC, NPAD), _f32),
        mesh=mesh,
        scratch_types=[
            pltpu.VMEM((NB, BB), jnp.int32),
            pltpu.VMEM((BB,), _f32),
            pltpu.VMEM((RPT,), _f32),
            pltpu.VMEM_SHARED((NPAD,), _f32),
        ],
    )
    prop = pl.kernel(
        _prop_body,
        out_type=jax.ShapeDtypeStruct((NC, NPAD, H), _f32),
        mesh=mesh,
        scratch_types=[
            pltpu.VMEM((NB, BB), jnp.int32),
            pltpu.VMEM((NB, BB), jnp.int32),
            pltpu.VMEM((BB, H), _f32),
            pltpu.VMEM((64, H), _f32),
            pltpu.VMEM_SHARED((NPAD, H), _f32),
        ],
    )
    return deg, prop


def _prop_body(y, srcp, dstp, outp, sidx, didx, rows, zb, acc_sp):
    c = lax.axis_index("c")
    s = lax.axis_index("s")
    wid = c * NS + s
    pltpu.sync_copy(srcp.at[wid], sidx)
    pltpu.sync_copy(dstp.at[wid], didx)

    def zf(i, _):
        for k in range(8):
            zb[i, pl.ds(k * 16, 16)] = jnp.zeros((16,), _f32)
        return 0
    lax.fori_loop(0, 64, zf, 0)

    def zcp(i, _):
        pltpu.sync_copy(zb, acc_sp.at[pl.ds(s * RPT + i * 64, 64)])
        return 0
    lax.fori_loop(0, RPT // 64, zcp, 0)
    plsc.subcore_barrier()

    def eb(j, _):
        pltpu.sync_copy(y.at[sidx.at[j]], rows)
        pltpu.sync_copy(rows, acc_sp.at[didx.at[j]], add=True)
        return 0
    lax.fori_loop(0, NB, eb, 0)
    plsc.subcore_barrier()
    pltpu.sync_copy(acc_sp.at[pl.ds(s * RPT, RPT)],
                    outp.at[c, pl.ds(s * RPT, RPT)])


BLK = 1000
NBLK = N // BLK


def _mm_body(x_ref, w_ref, o_ref):
    o_ref[...] = jnp.dot(x_ref[...], w_ref[...],
                         preferred_element_type=_f32)


_mm1 = pl.pallas_call(
    _mm_body,
    grid=(NBLK,),
    in_specs=[pl.BlockSpec((BLK, D), lambda i: (i, 0)),
              pl.BlockSpec((D, H), lambda i: (0, 0))],
    out_specs=pl.BlockSpec((BLK, H), lambda i: (i, 0)),
    out_shape=jax.ShapeDtypeStruct((N, H), _f32),
)


def _t0_body(degp_ref, xw_ref, dis_ref, dis2_ref, y_ref):
    deg = degp_ref[0] + degp_ref[1] + 1.0
    dis = lax.rsqrt(deg)
    dis2 = 1.0 / deg
    dis_ref[...] = dis
    dis2_ref[...] = dis2
    y_ref[...] = xw_ref[...] * dis


_t0 = pl.pallas_call(
    _t0_body,
    grid=(NBLK,),
    in_specs=[pl.BlockSpec((NC, BLK, 1), lambda i: (0, i, 0)),
              pl.BlockSpec((BLK, H), lambda i: (i, 0))],
    out_specs=[pl.BlockSpec((BLK, 1), lambda i: (i, 0)),
               pl.BlockSpec((BLK, 1), lambda i: (i, 0)),
               pl.BlockSpec((BLK, H), lambda i: (i, 0))],
    out_shape=[jax.ShapeDtypeStruct((N, 1), _f32),
               jax.ShapeDtypeStruct((N, 1), _f32),
               jax.ShapeDtypeStruct((N, H), _f32)],
)


def _blayer_body(with_next, with_cnt, *refs):
    if with_next:
        (acc_ref, xw_ref, dis_ref, dis2_ref, b_ref, bat_ref, w_ref,
         s_ref, *rest) = refs
        if with_cnt:
            cnt_ref, xwn_ref, yn_ref = rest
        else:
            xwn_ref, yn_ref = rest
    else:
        acc_ref, xw_ref, dis_ref, dis2_ref, b_ref, bat_ref, s_ref = refs
    i = pl.program_id(0)
    dis = dis_ref[...]
    a = acc_ref[0] + acc_ref[1]
    h = jnp.maximum(a * dis + xw_ref[...] * dis2_ref[...] + b_ref[...], 0.0)
    bat = bat_ref[0]
    gi = lax.broadcasted_iota(jnp.int32, (G, BLK), 0)
    oh = (gi == bat).astype(_f32)
    sc = jnp.dot(oh, h, preferred_element_type=_f32)

    @pl.when(i == 0)
    def _():
        s_ref[...] = jnp.zeros_like(s_ref)
        if with_next and with_cnt:
            cnt_ref[...] = jnp.zeros_like(cnt_ref)

    s_ref[...] += sc
    if with_next:
        if with_cnt:
            cnt_ref[...] += jnp.sum(oh, axis=1, keepdims=True)
        xwn = jnp.dot(h, w_ref[...], preferred_element_type=_f32)
        xwn_ref[...] = xwn
        yn_ref[...] = xwn * dis


def _make_blayer(with_next, with_cnt):
    in_specs = [
        pl.BlockSpec((NC, BLK, H), lambda i: (0, i, 0)),
        pl.BlockSpec((BLK, H), lambda i: (i, 0)),
        pl.BlockSpec((BLK, 1), lambda i: (i, 0)),
        pl.BlockSpec((BLK, 1), lambda i: (i, 0)),
        pl.BlockSpec((1, H), lambda i: (0, 0)),
        pl.BlockSpec((1, 1, BLK), lambda i: (i, 0, 0)),
    ]
    out_specs = [pl.BlockSpec((G, H), lambda i: (0, 0))]
    out_shape = [jax.ShapeDtypeStruct((G, H), _f32)]
    if with_next:
        in_specs.append(pl.BlockSpec((H, H), lambda i: (0, 0)))
        if with_cnt:
            out_specs.append(pl.BlockSpec((G, 1), lambda i: (0, 0)))
            out_shape.append(jax.ShapeDtypeStruct((G, 1), _f32))
        out_specs += [pl.BlockSpec((BLK, H), lambda i: (i, 0)),
                      pl.BlockSpec((BLK, H), lambda i: (i, 0))]
        out_shape += [jax.ShapeDtypeStruct((N, H), _f32),
                      jax.ShapeDtypeStruct((N, H), _f32)]
    return pl.pallas_call(
        functools.partial(_blayer_body, with_next, with_cnt),
        grid=(NBLK,),
        in_specs=in_specs,
        out_specs=out_specs,
        out_shape=out_shape,
    )


_b_first = _make_blayer(True, True)
_b_mid = _make_blayer(True, False)
_b_last = _make_blayer(False, False)


def _head_body(s1_ref, s2_ref, s3_ref, cnt_ref, pc_ref,
               lw1_ref, lb1_ref, lw2_ref, lb2_ref, lw3_ref, lb3_ref, o_ref):
    inv = 1.0 / jnp.maximum(cnt_ref[...], 1.0)
    g = (jnp.maximum(s1_ref[...] * inv, 0.0)
         + jnp.maximum(s2_ref[...] * inv, 0.0)
         + jnp.maximum(s3_ref[...] * inv, 0.0))
    g1 = jnp.maximum(
        jnp.dot(g, lw1_ref[...], preferred_element_type=_f32) + lb1_ref[...],
        0.0)
    l2 = lw2_ref[...]
    g2 = jnp.maximum(
        jnp.dot(g1, l2[:H // 2], preferred_element_type=_f32)
        + pc_ref[...] * l2[H // 2:H // 2 + 1]
        + lb2_ref[...],
        0.0)
    z = jnp.dot(g2, lw3_ref[...], preferred_element_type=_f32) + lb3_ref[...]
    m = jnp.max(z, axis=-1, keepdims=True)
    e = jnp.exp(z - m)
    o_ref[...] = z - m - jnp.log(jnp.sum(e, axis=-1, keepdims=True))


_head = pl.pallas_call(
    _head_body,
    grid=(1,),
    in_specs=[pl.BlockSpec((G, H), lambda i: (0, 0)),
              pl.BlockSpec((G, H), lambda i: (0, 0)),
              pl.BlockSpec((G, H), lambda i: (0, 0)),
              pl.BlockSpec((G, 1), lambda i: (0, 0)),
              pl.BlockSpec((G, 1), lambda i: (0, 0)),
              pl.BlockSpec((H, H // 2), lambda i: (0, 0)),
              pl.BlockSpec((1, H // 2), lambda i: (0, 0)),
              pl.BlockSpec((H // 2 + 1, H // 4), lambda i: (0, 0)),
              pl.BlockSpec((1, H // 4), lambda i: (0, 0)),
              pl.BlockSpec((H // 4, C), lambda i: (0, 0)),
              pl.BlockSpec((1, C), lambda i: (0, 0))],
    out_specs=pl.BlockSpec((G, C), lambda i: (0, 0)),
    out_shape=jax.ShapeDtypeStruct((G, C), _f32),
)


def kernel(x, edge_index, batch, paper_count, W1, b1, W2, b2, W3, b3,
           lw1, lb1, lw2, lb2, lw3, lb3):
    pad = EPAD - EPT
    src = edge_index[0].reshape(NW, EPT)
    dst = edge_index[1].reshape(NW, EPT)
    pad_dst = jnp.broadcast_to(
        TRASH + (jnp.arange(pad, dtype=jnp.int32) % (NPAD - N)), (NW, pad))
    pad_src = (jnp.arange(NW * pad, dtype=jnp.int32).reshape(NW, pad) * 41) % N
    srcp = jnp.concatenate([src, pad_src], axis=1).reshape(NW, NB, BB)
    dstp = jnp.concatenate(
        [dst, pad_dst], axis=1).reshape(NW, NB, BB)
    bat3 = batch.reshape(NBLK, 1, BLK)
    _deg_call, _prop_call = _sc_calls()

    degp = _deg_call(dstp).reshape(NC, NPAD, 1)
    xw1 = _mm1(x, W1)
    dis, dis2, y1 = _t0(degp, xw1)

    acc1 = _prop_call(y1, srcp, dstp)
    s1, cnt, xw2, y2 = _b_first(acc1, xw1, dis, dis2, b1.reshape(1, H),
                                bat3, W2)
    acc2 = _prop_call(y2, srcp, dstp)
    s2, xw3, y3 = _b_mid(acc2, xw2, dis, dis2, b2.reshape(1, H), bat3, W3)
    acc3 = _prop_call(y3, srcp, dstp)
    s3 = _b_last(acc3, xw3, dis, dis2, b3.reshape(1, H), bat3)
    if isinstance(s3, (list, tuple)):
        s3 = s3[0]

    return _head(s1, s2, s3, cnt, paper_count.reshape(G, 1),
                 lw1, lb1.reshape(1, H // 2), lw2, lb2.reshape(1, H // 4),
                 lw3, lb3.reshape(1, C))


# speedup vs baseline: 3.3756x; 1.1793x over previous
import functools

import jax
import jax.numpy as jnp
from jax import lax
from jax.experimental import pallas as pl
from jax.experimental.pallas import tpu as pltpu
from jax.experimental.pallas import tpu_sc as plsc

N = 10000
E = 320000
D = 128
H = 128
G = 64
C = 10

NC = 2
NS = 16
NW = NC * NS
EPT = E // NW
BB = 128
NB = 80
EPAD = NB * BB
NPAD = 10240
RPT = NPAD // NS
TRASH = N

_f32 = jnp.float32


def _deg_body(dstp, degp, idx_v, ones_v, zb_v, acc_sp):
    c = lax.axis_index("c")
    s = lax.axis_index("s")
    wid = c * NS + s
    pltpu.sync_copy(dstp.at[wid], idx_v)
    for k in range(8):
        ones_v[pl.ds(k * 16, 16)] = jnp.ones((16,), _f32)

    def zf(i, _):
        zb_v[pl.ds(i * 16, 16)] = jnp.zeros((16,), _f32)
        return 0
    lax.fori_loop(0, RPT // 16, zf, 0)
    pltpu.sync_copy(zb_v, acc_sp.at[pl.ds(s * RPT, RPT)])
    plsc.subcore_barrier()

    def eb(j, _):
        pltpu.sync_copy(ones_v, acc_sp.at[idx_v.at[j]], add=True)
        return 0
    lax.fori_loop(0, NB, eb, 0)
    plsc.subcore_barrier()
    pltpu.sync_copy(acc_sp.at[pl.ds(s * RPT, RPT)], degp.at[c, pl.ds(s * RPT, RPT)])


@functools.lru_cache(maxsize=None)
def _sc_calls():
    mesh = plsc.VectorSubcoreMesh(core_axis_name="c", subcore_axis_name="s")
    deg = pl.kernel(
        _deg_body,
        out_type=jax.ShapeDtypeStruct((NC, NPAD), _f32),
        mesh=mesh,
        scratch_types=[
            pltpu.VMEM((NB, BB), jnp.int32),
            pltpu.VMEM((BB,), _f32),
            pltpu.VMEM((RPT,), _f32),
            pltpu.VMEM_SHARED((NPAD,), _f32),
        ],
    )
    prop = pl.kernel(
        _prop_body,
        out_type=jax.ShapeDtypeStruct((NC, NPAD, H), _f32),
        mesh=mesh,
        scratch_types=[
            pltpu.VMEM((2, WS, BB), jnp.int32),
            pltpu.VMEM((NB, BB), jnp.int32),
            pltpu.VMEM((BB, H), _f32),
            pltpu.VMEM((BB, H), _f32),
            pltpu.VMEM((16, H), _f32),
            pltpu.SemaphoreType.DMA,
            pltpu.SemaphoreType.DMA,
            pltpu.SemaphoreType.DMA,
            pltpu.SemaphoreType.DMA,
            pltpu.VMEM_SHARED((NPAD, H), _f32),
        ],
    )
    return deg, prop


WS = 8
NWIN = NB // WS


def _prop_body(y, srcp, dstp, outp, swin, didx, rows0, rows1, zb,
               iw0, iw1, gs0, gs1, acc_sp):
    c = lax.axis_index("c")
    s = lax.axis_index("s")
    wid = c * NS + s
    iwsem = (iw0, iw1)

    def iw_start(w, bi):
        pltpu.async_copy(srcp.at[wid, pl.ds(w * WS, WS)], swin.at[bi],
                         iwsem[bi])

    def iw_wait(w, bi):
        pltpu.make_async_copy(srcp.at[wid, pl.ds(w * WS, WS)], swin.at[bi],
                              iwsem[bi]).wait()

    iw_start(0, 0)
    pltpu.sync_copy(dstp.at[wid], didx)

    def zf(i, _):
        for k in range(8):
            zb[i, pl.ds(k * 16, 16)] = jnp.zeros((16,), _f32)
        return 0
    lax.fori_loop(0, 16, zf, 0)

    def zcp(i, _):
        pltpu.sync_copy(zb, acc_sp.at[pl.ds(s * RPT + i * 16, 16)])
        return 0
    lax.fori_loop(0, RPT // 16, zcp, 0)
    plsc.subcore_barrier()

    def g_start(bi, lj, buf, sem):
        pltpu.async_copy(y.at[swin.at[bi, lj]], buf, sem)

    def g_drain(buf, sem):
        pltpu.make_async_copy(y.at[pl.ds(0, BB)], buf, sem).wait()

    def s_sync(jg, buf):
        pltpu.sync_copy(buf, acc_sp.at[didx.at[jg]], add=True)

    for w in range(NWIN):
        bi = w % 2
        base = w * WS
        if w + 1 < NWIN:
            iw_start(w + 1, 1 - bi)
        iw_wait(w, bi)
        g_start(bi, 0, rows0, gs0)

        def pair(k, _):
            lj0 = 2 * k
            lj1 = lj0 + 1
            ljn = jnp.minimum(lj1 + 1, WS - 1)
            g_drain(rows0, gs0)
            g_start(bi, lj1, rows1, gs1)
            s_sync(base + lj0, rows0)
            g_drain(rows1, gs1)
            g_start(bi, ljn, rows0, gs0)
            s_sync(base + lj1, rows1)
            return 0
        lax.fori_loop(0, WS // 2, pair, 0)
        g_drain(rows0, gs0)
    plsc.subcore_barrier()
    pltpu.sync_copy(acc_sp.at[pl.ds(s * RPT, RPT)],
                    outp.at[c, pl.ds(s * RPT, RPT)])


BLK = 1000
NBLK = N // BLK


def _mm_body(x_ref, w_ref, o_ref):
    o_ref[...] = jnp.dot(x_ref[...], w_ref[...],
                         preferred_element_type=_f32)


_mm1 = pl.pallas_call(
    _mm_body,
    grid=(NBLK,),
    in_specs=[pl.BlockSpec((BLK, D), lambda i: (i, 0)),
              pl.BlockSpec((D, H), lambda i: (0, 0))],
    out_specs=pl.BlockSpec((BLK, H), lambda i: (i, 0)),
    out_shape=jax.ShapeDtypeStruct((N, H), _f32),
)


def _t0_body(degp_ref, xw_ref, dis_ref, dis2_ref, y_ref):
    deg = degp_ref[0] + degp_ref[1] + 1.0
    dis = lax.rsqrt(deg)
    dis2 = 1.0 / deg
    dis_ref[...] = dis
    dis2_ref[...] = dis2
    y_ref[...] = xw_ref[...] * dis


_t0 = pl.pallas_call(
    _t0_body,
    grid=(NBLK,),
    in_specs=[pl.BlockSpec((NC, BLK, 1), lambda i: (0, i, 0)),
              pl.BlockSpec((BLK, H), lambda i: (i, 0))],
    out_specs=[pl.BlockSpec((BLK, 1), lambda i: (i, 0)),
               pl.BlockSpec((BLK, 1), lambda i: (i, 0)),
               pl.BlockSpec((BLK, H), lambda i: (i, 0))],
    out_shape=[jax.ShapeDtypeStruct((N, 1), _f32),
               jax.ShapeDtypeStruct((N, 1), _f32),
               jax.ShapeDtypeStruct((N, H), _f32)],
)


def _blayer_body(with_next, with_cnt, *refs):
    if with_next:
        (acc_ref, xw_ref, dis_ref, dis2_ref, b_ref, bat_ref, w_ref,
         s_ref, *rest) = refs
        if with_cnt:
            cnt_ref, xwn_ref, yn_ref = rest
        else:
            xwn_ref, yn_ref = rest
    else:
        acc_ref, xw_ref, dis_ref, dis2_ref, b_ref, bat_ref, s_ref = refs
    i = pl.program_id(0)
    dis = dis_ref[...]
    a = acc_ref[0] + acc_ref[1]
    h = jnp.maximum(a * dis + xw_ref[...] * dis2_ref[...] + b_ref[...], 0.0)
    bat = bat_ref[0]
    gi = lax.broadcasted_iota(jnp.int32, (G, BLK), 0)
    oh = (gi == bat).astype(_f32)
    sc = jnp.dot(oh, h, preferred_element_type=_f32)

    @pl.when(i == 0)
    def _():
        s_ref[...] = jnp.zeros_like(s_ref)
        if with_next and with_cnt:
            cnt_ref[...] = jnp.zeros_like(cnt_ref)

    s_ref[...] += sc
    if with_next:
        if with_cnt:
            cnt_ref[...] += jnp.sum(oh, axis=1, keepdims=True)
        xwn = jnp.dot(h, w_ref[...], preferred_element_type=_f32)
        xwn_ref[...] = xwn
        yn_ref[...] = xwn * dis


def _make_blayer(with_next, with_cnt):
    in_specs = [
        pl.BlockSpec((NC, BLK, H), lambda i: (0, i, 0)),
        pl.BlockSpec((BLK, H), lambda i: (i, 0)),
        pl.BlockSpec((BLK, 1), lambda i: (i, 0)),
        pl.BlockSpec((BLK, 1), lambda i: (i, 0)),
        pl.BlockSpec((1, H), lambda i: (0, 0)),
        pl.BlockSpec((1, 1, BLK), lambda i: (i, 0, 0)),
    ]
    out_specs = [pl.BlockSpec((G, H), lambda i: (0, 0))]
    out_shape = [jax.ShapeDtypeStruct((G, H), _f32)]
    if with_next:
        in_specs.append(pl.BlockSpec((H, H), lambda i: (0, 0)))
        if with_cnt:
            out_specs.append(pl.BlockSpec((G, 1), lambda i: (0, 0)))
            out_shape.append(jax.ShapeDtypeStruct((G, 1), _f32))
        out_specs += [pl.BlockSpec((BLK, H), lambda i: (i, 0)),
                      pl.BlockSpec((BLK, H), lambda i: (i, 0))]
        out_shape += [jax.ShapeDtypeStruct((N, H), _f32),
                      jax.ShapeDtypeStruct((N, H), _f32)]
    return pl.pallas_call(
        functools.partial(_blayer_body, with_next, with_cnt),
        grid=(NBLK,),
        in_specs=in_specs,
        out_specs=out_specs,
        out_shape=out_shape,
    )


_b_first = _make_blayer(True, True)
_b_mid = _make_blayer(True, False)
_b_last = _make_blayer(False, False)


def _head_body(s1_ref, s2_ref, s3_ref, cnt_ref, pc_ref,
               lw1_ref, lb1_ref, lw2_ref, lb2_ref, lw3_ref, lb3_ref, o_ref):
    inv = 1.0 / jnp.maximum(cnt_ref[...], 1.0)
    g = (jnp.maximum(s1_ref[...] * inv, 0.0)
         + jnp.maximum(s2_ref[...] * inv, 0.0)
         + jnp.maximum(s3_ref[...] * inv, 0.0))
    g1 = jnp.maximum(
        jnp.dot(g, lw1_ref[...], preferred_element_type=_f32) + lb1_ref[...],
        0.0)
    l2 = lw2_ref[...]
    g2 = jnp.maximum(
        jnp.dot(g1, l2[:H // 2], preferred_element_type=_f32)
        + pc_ref[...] * l2[H // 2:H // 2 + 1]
        + lb2_ref[...],
        0.0)
    z = jnp.dot(g2, lw3_ref[...], preferred_element_type=_f32) + lb3_ref[...]
    m = jnp.max(z, axis=-1, keepdims=True)
    e = jnp.exp(z - m)
    o_ref[...] = z - m - jnp.log(jnp.sum(e, axis=-1, keepdims=True))


_head = pl.pallas_call(
    _head_body,
    grid=(1,),
    in_specs=[pl.BlockSpec((G, H), lambda i: (0, 0)),
              pl.BlockSpec((G, H), lambda i: (0, 0)),
              pl.BlockSpec((G, H), lambda i: (0, 0)),
              pl.BlockSpec((G, 1), lambda i: (0, 0)),
              pl.BlockSpec((G, 1), lambda i: (0, 0)),
              pl.BlockSpec((H, H // 2), lambda i: (0, 0)),
              pl.BlockSpec((1, H // 2), lambda i: (0, 0)),
              pl.BlockSpec((H // 2 + 1, H // 4), lambda i: (0, 0)),
              pl.BlockSpec((1, H // 4), lambda i: (0, 0)),
              pl.BlockSpec((H // 4, C), lambda i: (0, 0)),
              pl.BlockSpec((1, C), lambda i: (0, 0))],
    out_specs=pl.BlockSpec((G, C), lambda i: (0, 0)),
    out_shape=jax.ShapeDtypeStruct((G, C), _f32),
)


def kernel(x, edge_index, batch, paper_count, W1, b1, W2, b2, W3, b3,
           lw1, lb1, lw2, lb2, lw3, lb3):
    pad = EPAD - EPT
    src = edge_index[0].reshape(NW, EPT)
    dst = edge_index[1].reshape(NW, EPT)
    pad_dst = jnp.broadcast_to(
        TRASH + (jnp.arange(pad, dtype=jnp.int32) % (NPAD - N)), (NW, pad))
    pad_src = (jnp.arange(NW * pad, dtype=jnp.int32).reshape(NW, pad) * 41) % N
    srcp = jnp.concatenate([src, pad_src], axis=1).reshape(NW, NB, BB)
    dstp = jnp.concatenate(
        [dst, pad_dst], axis=1).reshape(NW, NB, BB)
    bat3 = batch.reshape(NBLK, 1, BLK)
    _deg_call, _prop_call = _sc_calls()

    degp = _deg_call(dstp).reshape(NC, NPAD, 1)
    xw1 = _mm1(x, W1)
    dis, dis2, y1 = _t0(degp, xw1)

    acc1 = _prop_call(y1, srcp, dstp)
    s1, cnt, xw2, y2 = _b_first(acc1, xw1, dis, dis2, b1.reshape(1, H),
                                bat3, W2)
    acc2 = _prop_call(y2, srcp, dstp)
    s2, xw3, y3 = _b_mid(acc2, xw2, dis, dis2, b2.reshape(1, H), bat3, W3)
    acc3 = _prop_call(y3, srcp, dstp)
    s3 = _b_last(acc3, xw3, dis, dis2, b3.reshape(1, H), bat3)
    if isinstance(s3, (list, tuple)):
        s3 = s3[0]

    return _head(s1, s2, s3, cnt, paper_count.reshape(G, 1),
                 lw1, lb1.reshape(1, H // 2), lw2, lb2.reshape(1, H // 4),
                 lw3, lb3.reshape(1, C))


# speedup vs baseline: 3.5348x; 1.0472x over previous
import functools

import jax
import jax.numpy as jnp
from jax import lax
from jax.experimental import pallas as pl
from jax.experimental.pallas import tpu as pltpu
from jax.experimental.pallas import tpu_sc as plsc

N = 10000
E = 320000
D = 128
H = 128
G = 64
C = 10

NC = 2
NS = 16
NW = NC * NS
EPT = E // NW
BB = 128
NB = 80
EPAD = NB * BB
NPAD = 10240
RPT = NPAD // NS
TRASH = N

_f32 = jnp.float32


def _deg_body(dstp, degp, idx_v, ones_v, zb_v, acc_sp):
    c = lax.axis_index("c")
    s = lax.axis_index("s")
    wid = c * NS + s
    pltpu.sync_copy(dstp.at[wid], idx_v)
    for k in range(8):
        ones_v[pl.ds(k * 16, 16)] = jnp.ones((16,), _f32)

    def zf(i, _):
        zb_v[pl.ds(i * 16, 16)] = jnp.zeros((16,), _f32)
        return 0
    lax.fori_loop(0, RPT // 16, zf, 0)
    pltpu.sync_copy(zb_v, acc_sp.at[pl.ds(s * RPT, RPT)])
    plsc.subcore_barrier()

    def eb(j, _):
        pltpu.sync_copy(ones_v, acc_sp.at[idx_v.at[j]], add=True)
        return 0
    lax.fori_loop(0, NB, eb, 0)
    plsc.subcore_barrier()
    pltpu.sync_copy(acc_sp.at[pl.ds(s * RPT, RPT)], degp.at[c, pl.ds(s * RPT, RPT)])


@functools.lru_cache(maxsize=None)
def _sc_calls():
    mesh = plsc.VectorSubcoreMesh(core_axis_name="c", subcore_axis_name="s")
    deg = pl.kernel(
        _deg_body,
        out_type=jax.ShapeDtypeStruct((NC, NPAD), _f32),
        mesh=mesh,
        scratch_types=[
            pltpu.VMEM((NB, BB), jnp.int32),
            pltpu.VMEM((BB,), _f32),
            pltpu.VMEM((RPT,), _f32),
            pltpu.VMEM_SHARED((NPAD,), _f32),
        ],
    )
    prop = pl.kernel(
        _prop_body,
        out_type=jax.ShapeDtypeStruct((NC, NPAD, H), _f32),
        mesh=mesh,
        scratch_types=[
            pltpu.VMEM((2, WS, BB), jnp.int32),
            pltpu.VMEM((NB, BB), jnp.int32),
            pltpu.VMEM((BB, H), _f32),
            pltpu.VMEM((BB, H), _f32),
            pltpu.SemaphoreType.DMA,
            pltpu.SemaphoreType.DMA,
            pltpu.SemaphoreType.DMA,
            pltpu.SemaphoreType.DMA,
            pltpu.VMEM_SHARED((NPAD, H), _f32),
        ],
    )
    return deg, prop


WS = 16
NWIN = NB // WS


def _prop_body(y, srcp, dstp, outp, swin, didx, rows0, rows1,
               iw0, iw1, gs0, gs1, acc_sp):
    c = lax.axis_index("c")
    s = lax.axis_index("s")
    wid = c * NS + s
    iwsem = (iw0, iw1)

    def iw_start(w, bi):
        pltpu.async_copy(srcp.at[wid, pl.ds(w * WS, WS)], swin.at[bi],
                         iwsem[bi])

    def iw_wait(w, bi):
        pltpu.make_async_copy(srcp.at[wid, pl.ds(w * WS, WS)], swin.at[bi],
                              iwsem[bi]).wait()

    iw_start(0, 0)
    pltpu.sync_copy(dstp.at[wid], didx)

    def zf(i, _):
        for k in range(8):
            rows0[i, pl.ds(k * 16, 16)] = jnp.zeros((16,), _f32)
        return 0
    lax.fori_loop(0, BB, zf, 0)

    def zcp(i, _):
        pltpu.sync_copy(rows0, acc_sp.at[pl.ds(s * RPT + i * BB, BB)])
        return 0
    lax.fori_loop(0, RPT // BB, zcp, 0)
    plsc.subcore_barrier()

    def g_start(bi, lj, buf, sem):
        pltpu.async_copy(y.at[swin.at[bi, lj]], buf, sem)

    def g_drain(buf, sem):
        pltpu.make_async_copy(y.at[pl.ds(0, BB)], buf, sem).wait()

    def s_sync(jg, buf):
        pltpu.sync_copy(buf, acc_sp.at[didx.at[jg]], add=True)

    for w in range(NWIN):
        bi = w % 2
        base = w * WS
        if w + 1 < NWIN:
            iw_start(w + 1, 1 - bi)
        iw_wait(w, bi)
        g_start(bi, 0, rows0, gs0)

        def pair(k, _):
            lj0 = 2 * k
            lj1 = lj0 + 1
            ljn = jnp.minimum(lj1 + 1, WS - 1)
            g_drain(rows0, gs0)
            g_start(bi, lj1, rows1, gs1)
            s_sync(base + lj0, rows0)
            g_drain(rows1, gs1)
            g_start(bi, ljn, rows0, gs0)
            s_sync(base + lj1, rows1)
            return 0
        lax.fori_loop(0, WS // 2, pair, 0)
        g_drain(rows0, gs0)
    plsc.subcore_barrier()
    pltpu.sync_copy(acc_sp.at[pl.ds(s * RPT, RPT)],
                    outp.at[c, pl.ds(s * RPT, RPT)])


BLK = 1000
NBLK = N // BLK


def _t0_body(degp_ref, x_ref, w1_ref, dis_ref, dis2_ref, y_ref, xw_ref):
    xw = jnp.dot(x_ref[...], w1_ref[...], preferred_element_type=_f32)
    deg = degp_ref[0] + degp_ref[1] + 1.0
    dis = lax.rsqrt(deg)
    dis2 = 1.0 / deg
    dis_ref[...] = dis
    dis2_ref[...] = dis2
    y_ref[...] = xw * dis
    xw_ref[...] = xw


_t0 = pl.pallas_call(
    _t0_body,
    grid=(NBLK,),
    in_specs=[pl.BlockSpec((NC, BLK, 1), lambda i: (0, i, 0)),
              pl.BlockSpec((BLK, D), lambda i: (i, 0)),
              pl.BlockSpec((D, H), lambda i: (0, 0))],
    out_specs=[pl.BlockSpec((BLK, 1), lambda i: (i, 0)),
               pl.BlockSpec((BLK, 1), lambda i: (i, 0)),
               pl.BlockSpec((BLK, H), lambda i: (i, 0)),
               pl.BlockSpec((BLK, H), lambda i: (i, 0))],
    out_shape=[jax.ShapeDtypeStruct((N, 1), _f32),
               jax.ShapeDtypeStruct((N, 1), _f32),
               jax.ShapeDtypeStruct((N, H), _f32),
               jax.ShapeDtypeStruct((N, H), _f32)],
)


def _blayer_body(with_next, with_cnt, *refs):
    if with_next:
        (acc_ref, xw_ref, dis_ref, dis2_ref, b_ref, bat_ref, w_ref,
         s_ref, *rest) = refs
        if with_cnt:
            cnt_ref, xwn_ref, yn_ref = rest
        else:
            xwn_ref, yn_ref = rest
    else:
        acc_ref, xw_ref, dis_ref, dis2_ref, b_ref, bat_ref, s_ref = refs
    i = pl.program_id(0)
    dis = dis_ref[...]
    a = acc_ref[0] + acc_ref[1]
    h = jnp.maximum(a * dis + xw_ref[...] * dis2_ref[...] + b_ref[...], 0.0)
    bat = bat_ref[0]
    gi = lax.broadcasted_iota(jnp.int32, (G, BLK), 0)
    oh = (gi == bat).astype(_f32)
    sc = jnp.dot(oh, h, preferred_element_type=_f32)

    @pl.when(i == 0)
    def _():
        s_ref[...] = jnp.zeros_like(s_ref)
        if with_next and with_cnt:
            cnt_ref[...] = jnp.zeros_like(cnt_ref)

    s_ref[...] += sc
    if with_next:
        if with_cnt:
            cnt_ref[...] += jnp.sum(oh, axis=1, keepdims=True)
        xwn = jnp.dot(h, w_ref[...], preferred_element_type=_f32)
        xwn_ref[...] = xwn
        yn_ref[...] = xwn * dis


def _make_blayer(with_next, with_cnt):
    in_specs = [
        pl.BlockSpec((NC, BLK, H), lambda i: (0, i, 0)),
        pl.BlockSpec((BLK, H), lambda i: (i, 0)),
        pl.BlockSpec((BLK, 1), lambda i: (i, 0)),
        pl.BlockSpec((BLK, 1), lambda i: (i, 0)),
        pl.BlockSpec((1, H), lambda i: (0, 0)),
        pl.BlockSpec((1, 1, BLK), lambda i: (i, 0, 0)),
    ]
    out_specs = [pl.BlockSpec((G, H), lambda i: (0, 0))]
    out_shape = [jax.ShapeDtypeStruct((G, H), _f32)]
    if with_next:
        in_specs.append(pl.BlockSpec((H, H), lambda i: (0, 0)))
        if with_cnt:
            out_specs.append(pl.BlockSpec((G, 1), lambda i: (0, 0)))
            out_shape.append(jax.ShapeDtypeStruct((G, 1), _f32))
        out_specs += [pl.BlockSpec((BLK, H), lambda i: (i, 0)),
                      pl.BlockSpec((BLK, H), lambda i: (i, 0))]
        out_shape += [jax.ShapeDtypeStruct((N, H), _f32),
                      jax.ShapeDtypeStruct((N, H), _f32)]
    return pl.pallas_call(
        functools.partial(_blayer_body, with_next, with_cnt),
        grid=(NBLK,),
        in_specs=in_specs,
        out_specs=out_specs,
        out_shape=out_shape,
    )


_b_first = _make_blayer(True, True)
_b_mid = _make_blayer(True, False)


def _b3_body(acc_ref, xw_ref, dis_ref, dis2_ref, b_ref, bat_ref,
             s1_ref, s2_ref, cnt_ref, pc_ref,
             lw1_ref, lb1_ref, lw2_ref, lb2_ref, lw3_ref, lb3_ref,
             s_ref, o_ref):
    i = pl.program_id(0)
    dis = dis_ref[...]
    a = acc_ref[0] + acc_ref[1]
    h = jnp.maximum(a * dis + xw_ref[...] * dis2_ref[...] + b_ref[...], 0.0)
    bat = bat_ref[0]
    gi = lax.broadcasted_iota(jnp.int32, (G, BLK), 0)
    oh = (gi == bat).astype(_f32)
    sc = jnp.dot(oh, h, preferred_element_type=_f32)

    @pl.when(i == 0)
    def _():
        s_ref[...] = jnp.zeros_like(s_ref)

    s_ref[...] += sc

    @pl.when(i == NBLK - 1)
    def _():
        inv = 1.0 / jnp.maximum(cnt_ref[...], 1.0)
        g = (jnp.maximum(s1_ref[...] * inv, 0.0)
             + jnp.maximum(s2_ref[...] * inv, 0.0)
             + jnp.maximum(s_ref[...] * inv, 0.0))
        g1 = jnp.maximum(
            jnp.dot(g, lw1_ref[...], preferred_element_type=_f32)
            + lb1_ref[...], 0.0)
        l2 = lw2_ref[...]
        g2 = jnp.maximum(
            jnp.dot(g1, l2[:H // 2], preferred_element_type=_f32)
            + pc_ref[...] * l2[H // 2:H // 2 + 1]
            + lb2_ref[...], 0.0)
        z = (jnp.dot(g2, lw3_ref[...], preferred_element_type=_f32)
             + lb3_ref[...])
        m = jnp.max(z, axis=-1, keepdims=True)
        e = jnp.exp(z - m)
        o_ref[...] = z - m - jnp.log(jnp.sum(e, axis=-1, keepdims=True))


_b3 = pl.pallas_call(
    _b3_body,
    grid=(NBLK,),
    in_specs=[pl.BlockSpec((NC, BLK, H), lambda i: (0, i, 0)),
              pl.BlockSpec((BLK, H), lambda i: (i, 0)),
              pl.BlockSpec((BLK, 1), lambda i: (i, 0)),
              pl.BlockSpec((BLK, 1), lambda i: (i, 0)),
              pl.BlockSpec((1, H), lambda i: (0, 0)),
              pl.BlockSpec((1, 1, BLK), lambda i: (i, 0, 0)),
              pl.BlockSpec((G, H), lambda i: (0, 0)),
              pl.BlockSpec((G, H), lambda i: (0, 0)),
              pl.BlockSpec((G, 1), lambda i: (0, 0)),
              pl.BlockSpec((G, 1), lambda i: (0, 0)),
              pl.BlockSpec((H, H // 2), lambda i: (0, 0)),
              pl.BlockSpec((1, H // 2), lambda i: (0, 0)),
              pl.BlockSpec((H // 2 + 1, H // 4), lambda i: (0, 0)),
              pl.BlockSpec((1, H // 4), lambda i: (0, 0)),
              pl.BlockSpec((H // 4, C), lambda i: (0, 0)),
              pl.BlockSpec((1, C), lambda i: (0, 0))],
    out_specs=[pl.BlockSpec((G, H), lambda i: (0, 0)),
               pl.BlockSpec((G, C), lambda i: (0, 0))],
    out_shape=[jax.ShapeDtypeStruct((G, H), _f32),
               jax.ShapeDtypeStruct((G, C), _f32)],
)


def kernel(x, edge_index, batch, paper_count, W1, b1, W2, b2, W3, b3,
           lw1, lb1, lw2, lb2, lw3, lb3):
    pad = EPAD - EPT
    src = edge_index[0].reshape(NW, EPT)
    dst = edge_index[1].reshape(NW, EPT)
    pad_dst = jnp.broadcast_to(
        TRASH + (jnp.arange(pad, dtype=jnp.int32) % (NPAD - N)), (NW, pad))
    pad_src = (jnp.arange(NW * pad, dtype=jnp.int32).reshape(NW, pad) * 41) % N
    srcp = jnp.concatenate([src, pad_src], axis=1).reshape(NW, NB, BB)
    dstp = jnp.concatenate(
        [dst, pad_dst], axis=1).reshape(NW, NB, BB)
    bat3 = batch.reshape(NBLK, 1, BLK)
    _deg_call, _prop_call = _sc_calls()

    degp = _deg_call(dstp).reshape(NC, NPAD, 1)
    dis, dis2, y1, xw1 = _t0(degp, x, W1)

    acc1 = _prop_call(y1, srcp, dstp)
    s1, cnt, xw2, y2 = _b_first(acc1, xw1, dis, dis2, b1.reshape(1, H),
                                bat3, W2)
    acc2 = _prop_call(y2, srcp, dstp)
    s2, xw3, y3 = _b_mid(acc2, xw2, dis, dis2, b2.reshape(1, H), bat3, W3)
    acc3 = _prop_call(y3, srcp, dstp)
    _, out = _b3(acc3, xw3, dis, dis2, b3.reshape(1, H), bat3,
                 s1, s2, cnt, paper_count.reshape(G, 1),
                 lw1, lb1.reshape(1, H // 2), lw2, lb2.reshape(1, H // 4),
                 lw3, lb3.reshape(1, C))
    return out


# speedup vs baseline: 3.6603x; 1.0355x over previous
import functools

import jax
import jax.numpy as jnp
from jax import lax
from jax.experimental import pallas as pl
from jax.experimental.pallas import tpu as pltpu
from jax.experimental.pallas import tpu_sc as plsc

N = 10000
E = 320000
D = 128
H = 128
G = 64
C = 10

NC = 2
NS = 16
NW = NC * NS
EPT = E // NW
BB = 128
NB = 80
EPAD = NB * BB
NPAD = 10240
RPT = NPAD // NS
TRASH = N

_f32 = jnp.float32


def _deg_body(dstp, degp, idx_v, ones_v, zb_v, acc_sp):
    c = lax.axis_index("c")
    s = lax.axis_index("s")
    wid = c * NS + s
    pltpu.sync_copy(dstp.at[wid], idx_v)
    for k in range(8):
        ones_v[pl.ds(k * 16, 16)] = jnp.ones((16,), _f32)

    def zf(i, _):
        zb_v[pl.ds(i * 16, 16)] = jnp.zeros((16,), _f32)
        return 0
    lax.fori_loop(0, RPT // 16, zf, 0)
    pltpu.sync_copy(zb_v, acc_sp.at[pl.ds(s * RPT, RPT)])
    plsc.subcore_barrier()

    def eb(j, _):
        pltpu.sync_copy(ones_v, acc_sp.at[idx_v.at[j]], add=True)
        return 0
    lax.fori_loop(0, NB, eb, 0)
    plsc.subcore_barrier()
    pltpu.sync_copy(acc_sp.at[pl.ds(s * RPT, RPT)], degp.at[c, pl.ds(s * RPT, RPT)])


@functools.lru_cache(maxsize=None)
def _sc_calls():
    mesh = plsc.VectorSubcoreMesh(core_axis_name="c", subcore_axis_name="s")
    deg = pl.kernel(
        _deg_body,
        out_type=jax.ShapeDtypeStruct((NC, NPAD), _f32),
        mesh=mesh,
        scratch_types=[
            pltpu.VMEM((NB, BB), jnp.int32),
            pltpu.VMEM((BB,), _f32),
            pltpu.VMEM((RPT,), _f32),
            pltpu.VMEM_SHARED((NPAD,), _f32),
        ],
    )
    prop = pl.kernel(
        _prop_body,
        out_type=jax.ShapeDtypeStruct((NC, NPAD, H), _f32),
        mesh=mesh,
        scratch_types=[
            pltpu.VMEM((2, WS, BB), jnp.int32),
            pltpu.VMEM((NB, BB), jnp.int32),
            pltpu.VMEM((BB, H), _f32),
            pltpu.VMEM((BB, H), _f32),
            pltpu.SemaphoreType.DMA,
            pltpu.SemaphoreType.DMA,
            pltpu.SemaphoreType.DMA,
            pltpu.SemaphoreType.DMA,
            pltpu.VMEM_SHARED((NPAD, H), _f32),
        ],
    )
    return deg, prop


WS = 16
NWIN = NB // WS


def _prop_body(y, srcp, dstp, outp, swin, didx, rows0, rows1,
               iw0, iw1, gs0, gs1, acc_sp):
    c = lax.axis_index("c")
    s = lax.axis_index("s")
    wid = c * NS + s
    iwsem = (iw0, iw1)

    def iw_start(w, bi):
        pltpu.async_copy(srcp.at[wid, pl.ds(w * WS, WS)], swin.at[bi],
                         iwsem[bi])

    def iw_wait(w, bi):
        pltpu.make_async_copy(srcp.at[wid, pl.ds(w * WS, WS)], swin.at[bi],
                              iwsem[bi]).wait()

    iw_start(0, 0)
    pltpu.sync_copy(dstp.at[wid], didx)

    def zf(i, _):
        for k in range(8):
            rows0[i, pl.ds(k * 16, 16)] = jnp.zeros((16,), _f32)
        return 0
    lax.fori_loop(0, BB, zf, 0)

    def zcp(i, _):
        pltpu.sync_copy(rows0, acc_sp.at[pl.ds(s * RPT + i * BB, BB)])
        return 0
    lax.fori_loop(0, RPT // BB, zcp, 0)
    plsc.subcore_barrier()

    def g_start(bi, lj, buf, sem):
        pltpu.async_copy(y.at[swin.at[bi, lj]], buf, sem)

    def g_drain(buf, sem):
        pltpu.make_async_copy(y.at[pl.ds(0, BB)], buf, sem).wait()

    def s_sync(jg, buf):
        pltpu.sync_copy(buf, acc_sp.at[didx.at[jg]], add=True)

    iw_wait(0, 0)
    g_start(0, 0, rows0, gs0)
    for w in range(NWIN):
        bi = w % 2
        base = w * WS
        if w + 1 < NWIN:
            iw_start(w + 1, 1 - bi)

        def pair(k, _):
            lj0 = 2 * k
            lj1 = lj0 + 1
            g_drain(rows0, gs0)
            g_start(bi, lj1, rows1, gs1)
            s_sync(base + lj0, rows0)
            g_drain(rows1, gs1)
            g_start(bi, lj1 + 1, rows0, gs0)
            s_sync(base + lj1, rows1)
            return 0
        lax.fori_loop(0, WS // 2 - 1, pair, 0)
        g_drain(rows0, gs0)
        g_start(bi, WS - 1, rows1, gs1)
        s_sync(base + WS - 2, rows0)
        g_drain(rows1, gs1)
        if w + 1 < NWIN:
            iw_wait(w + 1, 1 - bi)
            g_start(1 - bi, 0, rows0, gs0)
        s_sync(base + WS - 1, rows1)
    plsc.subcore_barrier()
    pltpu.sync_copy(acc_sp.at[pl.ds(s * RPT, RPT)],
                    outp.at[c, pl.ds(s * RPT, RPT)])


BLK = 1000
NBLK = N // BLK


def _t0_body(degp_ref, x_ref, w1_ref, dis_ref, dis2_ref, y_ref, xw_ref):
    xw = jnp.dot(x_ref[...], w1_ref[...], preferred_element_type=_f32)
    deg = degp_ref[0] + degp_ref[1] + 1.0
    dis = lax.rsqrt(deg)
    dis2 = 1.0 / deg
    dis_ref[...] = dis
    dis2_ref[...] = dis2
    y_ref[...] = xw * dis
    xw_ref[...] = xw


_t0 = pl.pallas_call(
    _t0_body,
    grid=(NBLK,),
    in_specs=[pl.BlockSpec((NC, BLK, 1), lambda i: (0, i, 0)),
              pl.BlockSpec((BLK, D), lambda i: (i, 0)),
              pl.BlockSpec((D, H), lambda i: (0, 0))],
    out_specs=[pl.BlockSpec((BLK, 1), lambda i: (i, 0)),
               pl.BlockSpec((BLK, 1), lambda i: (i, 0)),
               pl.BlockSpec((BLK, H), lambda i: (i, 0)),
               pl.BlockSpec((BLK, H), lambda i: (i, 0))],
    out_shape=[jax.ShapeDtypeStruct((N, 1), _f32),
               jax.ShapeDtypeStruct((N, 1), _f32),
               jax.ShapeDtypeStruct((N, H), _f32),
               jax.ShapeDtypeStruct((N, H), _f32)],
)


def _blayer_body(with_next, with_cnt, *refs):
    if with_next:
        (acc_ref, xw_ref, dis_ref, dis2_ref, b_ref, bat_ref, w_ref,
         s_ref, *rest) = refs
        if with_cnt:
            cnt_ref, xwn_ref, yn_ref = rest
        else:
            xwn_ref, yn_ref = rest
    else:
        acc_ref, xw_ref, dis_ref, dis2_ref, b_ref, bat_ref, s_ref = refs
    i = pl.program_id(0)
    dis = dis_ref[...]
    a = acc_ref[0] + acc_ref[1]
    h = jnp.maximum(a * dis + xw_ref[...] * dis2_ref[...] + b_ref[...], 0.0)
    bat = bat_ref[0]
    gi = lax.broadcasted_iota(jnp.int32, (G, BLK), 0)
    oh = (gi == bat).astype(_f32)
    sc = jnp.dot(oh, h, preferred_element_type=_f32)

    @pl.when(i == 0)
    def _():
        s_ref[...] = jnp.zeros_like(s_ref)
        if with_next and with_cnt:
            cnt_ref[...] = jnp.zeros_like(cnt_ref)

    s_ref[...] += sc
    if with_next:
        if with_cnt:
            cnt_ref[...] += jnp.sum(oh, axis=1, keepdims=True)
        xwn = jnp.dot(h, w_ref[...], preferred_element_type=_f32)
        xwn_ref[...] = xwn
        yn_ref[...] = xwn * dis


def _make_blayer(with_next, with_cnt):
    in_specs = [
        pl.BlockSpec((NC, BLK, H), lambda i: (0, i, 0)),
        pl.BlockSpec((BLK, H), lambda i: (i, 0)),
        pl.BlockSpec((BLK, 1), lambda i: (i, 0)),
        pl.BlockSpec((BLK, 1), lambda i: (i, 0)),
        pl.BlockSpec((1, H), lambda i: (0, 0)),
        pl.BlockSpec((1, 1, BLK), lambda i: (i, 0, 0)),
    ]
    out_specs = [pl.BlockSpec((G, H), lambda i: (0, 0))]
    out_shape = [jax.ShapeDtypeStruct((G, H), _f32)]
    if with_next:
        in_specs.append(pl.BlockSpec((H, H), lambda i: (0, 0)))
        if with_cnt:
            out_specs.append(pl.BlockSpec((G, 1), lambda i: (0, 0)))
            out_shape.append(jax.ShapeDtypeStruct((G, 1), _f32))
        out_specs += [pl.BlockSpec((BLK, H), lambda i: (i, 0)),
                      pl.BlockSpec((BLK, H), lambda i: (i, 0))]
        out_shape += [jax.ShapeDtypeStruct((N, H), _f32),
                      jax.ShapeDtypeStruct((N, H), _f32)]
    return pl.pallas_call(
        functools.partial(_blayer_body, with_next, with_cnt),
        grid=(NBLK,),
        in_specs=in_specs,
        out_specs=out_specs,
        out_shape=out_shape,
    )


_b_first = _make_blayer(True, True)
_b_mid = _make_blayer(True, False)


def _b3_body(acc_ref, xw_ref, dis_ref, dis2_ref, b_ref, bat_ref,
             s1_ref, s2_ref, cnt_ref, pc_ref,
             lw1_ref, lb1_ref, lw2_ref, lb2_ref, lw3_ref, lb3_ref,
             s_ref, o_ref):
    i = pl.program_id(0)
    dis = dis_ref[...]
    a = acc_ref[0] + acc_ref[1]
    h = jnp.maximum(a * dis + xw_ref[...] * dis2_ref[...] + b_ref[...], 0.0)
    bat = bat_ref[0]
    gi = lax.broadcasted_iota(jnp.int32, (G, BLK), 0)
    oh = (gi == bat).astype(_f32)
    sc = jnp.dot(oh, h, preferred_element_type=_f32)

    @pl.when(i == 0)
    def _():
        s_ref[...] = jnp.zeros_like(s_ref)

    s_ref[...] += sc

    @pl.when(i == NBLK - 1)
    def _():
        inv = 1.0 / jnp.maximum(cnt_ref[...], 1.0)
        g = (jnp.maximum(s1_ref[...] * inv, 0.0)
             + jnp.maximum(s2_ref[...] * inv, 0.0)
             + jnp.maximum(s_ref[...] * inv, 0.0))
        g1 = jnp.maximum(
            jnp.dot(g, lw1_ref[...], preferred_element_type=_f32)
            + lb1_ref[...], 0.0)
        l2 = lw2_ref[...]
        g2 = jnp.maximum(
            jnp.dot(g1, l2[:H // 2], preferred_element_type=_f32)
            + pc_ref[...] * l2[H // 2:H // 2 + 1]
            + lb2_ref[...], 0.0)
        z = (jnp.dot(g2, lw3_ref[...], preferred_element_type=_f32)
             + lb3_ref[...])
        m = jnp.max(z, axis=-1, keepdims=True)
        e = jnp.exp(z - m)
        o_ref[...] = z - m - jnp.log(jnp.sum(e, axis=-1, keepdims=True))


_b3 = pl.pallas_call(
    _b3_body,
    grid=(NBLK,),
    in_specs=[pl.BlockSpec((NC, BLK, H), lambda i: (0, i, 0)),
              pl.BlockSpec((BLK, H), lambda i: (i, 0)),
              pl.BlockSpec((BLK, 1), lambda i: (i, 0)),
              pl.BlockSpec((BLK, 1), lambda i: (i, 0)),
              pl.BlockSpec((1, H), lambda i: (0, 0)),
              pl.BlockSpec((1, 1, BLK), lambda i: (i, 0, 0)),
              pl.BlockSpec((G, H), lambda i: (0, 0)),
              pl.BlockSpec((G, H), lambda i: (0, 0)),
              pl.BlockSpec((G, 1), lambda i: (0, 0)),
              pl.BlockSpec((G, 1), lambda i: (0, 0)),
              pl.BlockSpec((H, H // 2), lambda i: (0, 0)),
              pl.BlockSpec((1, H // 2), lambda i: (0, 0)),
              pl.BlockSpec((H // 2 + 1, H // 4), lambda i: (0, 0)),
              pl.BlockSpec((1, H // 4), lambda i: (0, 0)),
              pl.BlockSpec((H // 4, C), lambda i: (0, 0)),
              pl.BlockSpec((1, C), lambda i: (0, 0))],
    out_specs=[pl.BlockSpec((G, H), lambda i: (0, 0)),
               pl.BlockSpec((G, C), lambda i: (0, 0))],
    out_shape=[jax.ShapeDtypeStruct((G, H), _f32),
               jax.ShapeDtypeStruct((G, C), _f32)],
)


def kernel(x, edge_index, batch, paper_count, W1, b1, W2, b2, W3, b3,
           lw1, lb1, lw2, lb2, lw3, lb3):
    pad = EPAD - EPT
    src = edge_index[0].reshape(NW, EPT)
    dst = edge_index[1].reshape(NW, EPT)
    pad_dst = jnp.broadcast_to(
        TRASH + (jnp.arange(pad, dtype=jnp.int32) % (NPAD - N)), (NW, pad))
    pad_src = (jnp.arange(NW * pad, dtype=jnp.int32).reshape(NW, pad) * 41) % N
    srcp = jnp.concatenate([src, pad_src], axis=1).reshape(NW, NB, BB)
    dstp = jnp.concatenate(
        [dst, pad_dst], axis=1).reshape(NW, NB, BB)
    bat3 = batch.reshape(NBLK, 1, BLK)
    _deg_call, _prop_call = _sc_calls()

    degp = _deg_call(dstp).reshape(NC, NPAD, 1)
    dis, dis2, y1, xw1 = _t0(degp, x, W1)

    acc1 = _prop_call(y1, srcp, dstp)
    s1, cnt, xw2, y2 = _b_first(acc1, xw1, dis, dis2, b1.reshape(1, H),
                                bat3, W2)
    acc2 = _prop_call(y2, srcp, dstp)
    s2, xw3, y3 = _b_mid(acc2, xw2, dis, dis2, b2.reshape(1, H), bat3, W3)
    acc3 = _prop_call(y3, srcp, dstp)
    _, out = _b3(acc3, xw3, dis, dis2, b3.reshape(1, H), bat3,
                 s1, s2, cnt, paper_count.reshape(G, 1),
                 lw1, lb1.reshape(1, H // 2), lw2, lb2.reshape(1, H // 4),
                 lw3, lb3.reshape(1, C))
    return out
